# baseline (speedup 1.0000x reference)
import jax
import jax.numpy as jnp
from jax.experimental import pallas as pl
from jax.experimental.pallas import tpu as pltpu

N_ROWS = 100000
T_COLS = 256
N_TGT = 128
CHUNK = 4000
N_CHUNKS = N_ROWS // CHUNK


def _body(sorted_t_ref, order_ref, nuniq_ref, stim_ref, o_ref,
          z_ref, rows_ref, zsem, rsem):
    z_ref[...] = jnp.zeros_like(z_ref)
    for c in range(N_CHUNKS):
        pltpu.make_async_copy(
            z_ref, o_ref.at[pl.ds(c * CHUNK, CHUNK), :], zsem).start()

    def combine(j, carry):
        i = order_ref[j]
        dup = jnp.logical_and(j > 0, sorted_t_ref[j] == sorted_t_ref[j - 1])
        prev = jnp.where(dup, rows_ref[pl.ds(j - 1, 1), :], 0.0)
        rows_ref[pl.ds(j, 1), :] = stim_ref[pl.ds(i, 1), :] + prev
        return carry

    jax.lax.fori_loop(0, N_TGT, combine, 0)

    for c in range(N_CHUNKS):
        pltpu.make_async_copy(
            z_ref, o_ref.at[pl.ds(c * CHUNK, CHUNK), :], zsem).wait()

    def fire(j, carry):
        t = sorted_t_ref[j]
        last = jnp.logical_or(j == N_TGT - 1, sorted_t_ref[j + 1] != t)

        @pl.when(last)
        def _():
            pltpu.make_async_copy(
                rows_ref.at[pl.ds(j, 1), :],
                o_ref.at[pl.ds(t, 1), :], rsem).start()

        return carry

    jax.lax.fori_loop(0, N_TGT, fire, 0)

    def drain(j, carry):
        pltpu.make_async_copy(
            rows_ref.at[pl.ds(0, 1), :], o_ref.at[pl.ds(0, 1), :], rsem).wait()
        return carry

    jax.lax.fori_loop(0, nuniq_ref[0], drain, 0)


def kernel(stimuli, targets):
    tgt = targets.astype(jnp.int32)
    order = jnp.argsort(tgt).astype(jnp.int32)
    sorted_t = tgt[order]
    nuniq = (jnp.sum(sorted_t[1:] != sorted_t[:-1]) + 1).astype(jnp.int32)

    grid_spec = pltpu.PrefetchScalarGridSpec(
        num_scalar_prefetch=3,
        grid=(1,),
        in_specs=[
            pl.BlockSpec((N_TGT, T_COLS), lambda b, *_: (0, 0)),
        ],
        out_specs=pl.BlockSpec(memory_space=pl.ANY),
        scratch_shapes=[
            pltpu.VMEM((CHUNK, T_COLS), jnp.float32),
            pltpu.VMEM((N_TGT, T_COLS), jnp.float32),
            pltpu.SemaphoreType.DMA,
            pltpu.SemaphoreType.DMA,
        ],
    )
    return pl.pallas_call(
        _body,
        grid_spec=grid_spec,
        out_shape=jax.ShapeDtypeStruct((N_ROWS, T_COLS), jnp.float32),
    )(sorted_t, order, nuniq.reshape(1), stimuli)
